# Initial kernel scaffold; baseline (speedup 1.0000x reference)
import jax
import jax.numpy as jnp
from jax.experimental import pallas as pl


def kernel(x, edge_index, W1, b1, W2, b2, W3, b3, in_proj_w, in_proj_b, out_proj_w, out_proj_b):
    raise NotImplementedError("write your pallas kernel here")


# speedup vs baseline: 2.2633x; 2.2633x over previous
import functools
import math

import jax
import jax.numpy as jnp
from jax import lax
from jax.experimental import pallas as pl
from jax.experimental.pallas import tpu as pltpu
from jax.experimental.pallas import tpu_sc as plsc

N = 4096
E = 131072
C = 256
HEADS = 8
HD = 32
NW = 32
EPW = E // NW
NB = 64
LANE = 16
INV_SQRT_HD = 1.0 / math.sqrt(HD)
BIG = 1e30

_mesh = plsc.VectorSubcoreMesh(core_axis_name="c", subcore_axis_name="s")
_sc_params = pltpu.CompilerParams(needs_layout_passes=False)


def _wid():
    return lax.axis_index("s") * 2 + lax.axis_index("c")


def _lane_iota():
    return lax.broadcasted_iota(jnp.int32, (LANE,), 0)


def _splat_last(v):
    idx = jnp.full((LANE, 1), LANE - 1, jnp.int32)
    dn = lax.GatherDimensionNumbers(
        offset_dims=(), collapsed_slice_dims=(0,), start_index_map=(0,))
    return lax.gather(v, idx, dn, (1,),
                      mode=lax.GatherScatterMode.PROMISE_IN_BOUNDS)


def _splat_lane(v, j):
    idx = jnp.full((LANE, 1), j, jnp.int32)
    dn = lax.GatherDimensionNumbers(
        offset_dims=(), collapsed_slice_dims=(0,), start_index_map=(0,))
    return lax.gather(v, idx, dn, (1,),
                      mode=lax.GatherScatterMode.PROMISE_IN_BOUNDS)


def _sc_hist(weighted):
    scratch = [
        pltpu.VMEM((N,), jnp.float32),
        pltpu.VMEM((EPW,), jnp.int32),
    ]
    if weighted:
        scratch.append(pltpu.VMEM((EPW,), jnp.float32))

    @functools.partial(
        pl.kernel,
        out_type=jax.ShapeDtypeStruct((NW, N), jnp.float32),
        mesh=_mesh,
        scratch_types=scratch,
        compiler_params=_sc_params,
    )
    def hist(*refs):
        if weighted:
            dst_hbm, w_hbm, out_hbm, hist_v, dst_v, w_v = refs
        else:
            dst_hbm, out_hbm, hist_v, dst_v = refs
        wid = _wid()
        base = wid * EPW
        pltpu.sync_copy(dst_hbm.at[pl.ds(base, EPW)], dst_v)
        if weighted:
            pltpu.sync_copy(w_hbm.at[pl.ds(base, EPW)], w_v)

        def zero_body(i, _):
            hist_v[pl.ds(i * LANE, LANE)] = jnp.zeros((LANE,), jnp.float32)
            return 0
        lax.fori_loop(0, N // LANE, zero_body, 0)

        ones = jnp.ones((LANE,), jnp.float32)

        def acc_body(i, _):
            d = dst_v[pl.ds(i * LANE, LANE)]
            w = w_v[pl.ds(i * LANE, LANE)] if weighted else ones
            plsc.addupdate_scatter(hist_v, [d], w)
            return 0
        lax.fori_loop(0, EPW // LANE, acc_body, 0)

        pltpu.sync_copy(hist_v, out_hbm.at[wid])

    return hist


NBE = 16


AW = 2 * C + 128


@functools.partial(
    pl.kernel,
    out_type=jax.ShapeDtypeStruct((E * LANE,), jnp.float32),
    mesh=_mesh,
    scratch_types=[
        pltpu.VMEM((EPW,), jnp.int32),
        pltpu.VMEM((EPW,), jnp.int32),
        pltpu.VMEM((NBE, AW), jnp.float32),
        pltpu.VMEM((NBE, 2 * C), jnp.float32),
        pltpu.VMEM((EPW * LANE,), jnp.float32),
        pltpu.SemaphoreType.DMA,
    ],
    compiler_params=_sc_params,
)
def _sc_edge(a_hbm, b_hbm, src_hbm, dst_hbm, z_hbm,
             src_v, dst_v, a_v, b_v, zb_v, sem):
    wid = _wid()
    base = wid * EPW
    pltpu.sync_copy(src_hbm.at[pl.ds(base, EPW)], src_v)
    pltpu.sync_copy(dst_hbm.at[pl.ds(base, EPW)], dst_v)

    lane = _lane_iota()

    def batch_body(bi, _):
        eb = bi * NBE
        sv = src_v[pl.ds(eb, LANE)]
        dv = dst_v[pl.ds(eb, LANE)]

        def edge_body(e, _):
            z = jnp.full((LANE,), 0.5, jnp.float32)
            zb_v[pl.ds((eb + e) * LANE, LANE)] = z
            return 0
        lax.fori_loop(0, NBE, edge_body, 0)
        return 0

    lax.fori_loop(0, EPW // NBE, batch_body, 0)
    pltpu.sync_copy(zb_v, z_hbm.at[pl.ds(base * LANE, EPW * LANE)])


def _ewtc_body(z_ref, o_ref):
    z = z_ref[...]
    col = lax.broadcasted_iota(jnp.int32, z.shape, 1)
    es = jnp.sum(jnp.where(col < HEADS, jnp.exp(z), 0.0), axis=1,
                 keepdims=True)
    sim = jnp.sum(jnp.where(col == (LANE - 1), z, 0.0), axis=1, keepdims=True)
    o_ref[...] = sim * es * (1.0 / HEADS)


def _ew_from_z(zflat, bm=8192):
    z2 = zflat.reshape(E, LANE)
    out = pl.pallas_call(
        _ewtc_body,
        grid=(E // bm,),
        in_specs=[pl.BlockSpec((bm, LANE), lambda i: (i, 0))],
        out_specs=pl.BlockSpec((bm, 1), lambda i: (i, 0)),
        out_shape=jax.ShapeDtypeStruct((E, 1), jnp.float32),
    )(z2)
    return out.reshape(E)


ROWS_PT = N // NW
NBM = 1024


def _sc_msg(weighted):
    scratch = [
        pltpu.VMEM((ROWS_PT + 1, C), jnp.float32),
        pltpu.VMEM((NBM,), jnp.int32),
        pltpu.VMEM((NBM,), jnp.int32),
        pltpu.VMEM((NBM + LANE,), jnp.int32),
        pltpu.VMEM((NBM + LANE,), jnp.int32),
        pltpu.VMEM((LANE, C), jnp.float32),
        pltpu.SemaphoreType.DMA,
    ]
    if weighted:
        scratch.insert(3, pltpu.VMEM((NBM,), jnp.float32))
        scratch.insert(6, pltpu.VMEM((NBM + LANE,), jnp.float32))

    @functools.partial(
        pl.kernel,
        out_type=jax.ShapeDtypeStruct((N, C), jnp.float32),
        mesh=_mesh,
        scratch_types=scratch,
        compiler_params=_sc_params,
    )
    def msg(*refs):
        if weighted:
            (xw_hbm, src_hbm, dst_hbm, w_hbm, out_hbm,
             acc_v, src_v, dst_v, w_v, csrc_v, cdst_v, cw_v,
             rows_v, sem) = refs
        else:
            (xw_hbm, src_hbm, dst_hbm, out_hbm,
             acc_v, src_v, dst_v, csrc_v, cdst_v, rows_v, sem) = refs
        wid = _wid()
        lo = wid * ROWS_PT
        lane = _lane_iota()

        def zrow(i, _):
            def zcol(j, _):
                acc_v[i, pl.ds(j * LANE, LANE)] = jnp.zeros((LANE,),
                                                            jnp.float32)
                return 0
            lax.fori_loop(0, C // LANE, zcol, 0)
            return 0
        lax.fori_loop(0, ROWS_PT + 1, zrow, 0)

        lo_v = jnp.full((LANE,), lo, jnp.int32)
        hi_v = jnp.full((LANE,), lo + ROWS_PT, jnp.int32)

        def batch_body(bi, _):
            eb = bi * NBM
            pltpu.sync_copy(src_hbm.at[pl.ds(eb, NBM)], src_v)
            pltpu.sync_copy(dst_hbm.at[pl.ds(eb, NBM)], dst_v)
            if weighted:
                pltpu.sync_copy(w_hbm.at[pl.ds(eb, NBM)], w_v)

            def scan_g(g, cnt):
                d = dst_v[pl.ds(g * LANE, LANE)]
                m = (d >= lo_v) & (d < hi_v)
                s = src_v[pl.ds(g * LANE, LANE)]
                plsc.store_compressed(csrc_v.at[pl.ds(cnt, LANE)], s, mask=m)
                plsc.store_compressed(cdst_v.at[pl.ds(cnt, LANE)],
                                      d - lo_v, mask=m)
                if weighted:
                    w = w_v[pl.ds(g * LANE, LANE)]
                    plsc.store_compressed(cw_v.at[pl.ds(cnt, LANE)], w, mask=m)
                return cnt + jnp.sum(m.astype(jnp.int32))
            cnt = lax.fori_loop(0, NBM // LANE, scan_g, 0)

            def drain_g(t, _):
                rem = cnt - t * LANE
                valid = lane < jnp.full((LANE,), rem, jnp.int32)
                idxs = csrc_v[pl.ds(t * LANE, LANE)]
                pltpu.async_copy(xw_hbm.at[idxs], rows_v, sem).wait()
                dloc = jnp.where(valid, cdst_v[pl.ds(t * LANE, LANE)],
                                 ROWS_PT)
                if weighted:
                    wch = cw_v[pl.ds(t * LANE, LANE)]
                for j in range(LANE):
                    sel = (lane == j).astype(jnp.int32)
                    dl = jnp.sum(sel * dloc)
                    if weighted:
                        ws = jnp.sum(sel.astype(jnp.float32) * wch)
                    for cix in range(C // LANE):
                        sl = pl.ds(cix * LANE, LANE)
                        chunk = rows_v[j, sl]
                        if weighted:
                            chunk = chunk * ws
                        plsc.addupdate(acc_v.at[dl, sl], chunk)
                return 0
            lax.fori_loop(0, (cnt + LANE - 1) // LANE, drain_g, 0)
            return 0

        lax.fori_loop(0, E // NBM, batch_body, 0)
        pltpu.sync_copy(acc_v.at[pl.ds(0, ROWS_PT)],
                        out_hbm.at[pl.ds(lo, ROWS_PT)])

    return msg


def _mm_body(x_ref, w_ref, *rest, bias, rowscale):
    y = jnp.dot(x_ref[...], w_ref[...], preferred_element_type=jnp.float32)
    i = 0
    if bias:
        y = y + rest[i][...]
        i += 1
    if rowscale:
        y = y * rest[i][...]
        i += 1
    rest[i][...] = y


def _matmul(x, w, b=None, rowscale=None, bm=512):
    m, k = x.shape
    n = w.shape[1]
    args = [x, w]
    in_specs = [
        pl.BlockSpec((bm, k), lambda i: (i, 0)),
        pl.BlockSpec((k, n), lambda i: (0, 0)),
    ]
    if b is not None:
        args.append(b.reshape(1, n))
        in_specs.append(pl.BlockSpec((1, n), lambda i: (0, 0)))
    if rowscale is not None:
        args.append(rowscale)
        in_specs.append(pl.BlockSpec((bm, 1), lambda i: (i, 0)))
    return pl.pallas_call(
        functools.partial(_mm_body, bias=b is not None,
                          rowscale=rowscale is not None),
        grid=(m // bm,),
        in_specs=in_specs,
        out_specs=pl.BlockSpec((bm, n), lambda i: (i, 0)),
        out_shape=jax.ShapeDtypeStruct((m, n), jnp.float32),
    )(*args)


def _lse_body(q_ref, k_ref, o_ref, *, bq):
    i = pl.program_id(1)
    q = q_ref[0]
    k = k_ref[0]
    s = lax.dot_general(q, k, (((1,), (1,)), ((), ())),
                        preferred_element_type=jnp.float32) * INV_SQRT_HD
    m = jnp.max(s, axis=1, keepdims=True)
    lse = m[:, 0] + jnp.log(jnp.sum(jnp.exp(s - m), axis=1))
    o_ref[0, 0, pl.ds(i * bq, bq)] = lse


def _flash_lse(q3, k3, bq=256):
    out = pl.pallas_call(
        functools.partial(_lse_body, bq=bq),
        grid=(HEADS, N // bq),
        in_specs=[
            pl.BlockSpec((1, bq, HD), lambda h, i: (h, i, 0)),
            pl.BlockSpec((1, N, HD), lambda h, i: (h, 0, 0)),
        ],
        out_specs=pl.BlockSpec((1, 1, N), lambda h, i: (h, 0, 0)),
        out_shape=jax.ShapeDtypeStruct((HEADS, 1, N), jnp.float32),
    )(q3, k3)
    return out.reshape(HEADS, N)


def _deg_body(h_ref, o_ref):
    deg = 1.0 + jnp.sum(h_ref[...], axis=0, keepdims=True)
    o_ref[...] = jnp.where(deg > 0, lax.rsqrt(jnp.where(deg > 0, deg, 1.0)),
                           0.0)


def _deg_to_dinv(hist):
    return pl.pallas_call(
        _deg_body,
        in_specs=[pl.BlockSpec((1, N), lambda: (0, 0))],
        out_specs=pl.BlockSpec((1, N), lambda: (0, 0)),
        out_shape=jax.ShapeDtypeStruct((1, N), jnp.float32),
    )(hist)


def _epi_body(a0_ref, xwp_ref, dinv_ref, b_ref, o_ref, *norm_out,
              relu, emit_norm):
    y = dinv_ref[...] * (a0_ref[...] + xwp_ref[...]) + b_ref[...]
    if relu:
        y = jnp.maximum(y, 0.0)
    o_ref[...] = y
    if emit_norm:
        n2 = jnp.sum(y * y, axis=1, keepdims=True)
        inv = 1.0 / jnp.maximum(jnp.sqrt(n2), 1e-8)
        norm_out[0][...] = y * inv


def _epilogue(a0, xwp, dinv_col, b, relu, emit_norm=False, bm=512):
    out_shape = jax.ShapeDtypeStruct((N, C), jnp.float32)
    out_specs = pl.BlockSpec((bm, C), lambda i: (i, 0))
    if emit_norm:
        out_shape = (out_shape, jax.ShapeDtypeStruct((N, C), jnp.float32))
        out_specs = (out_specs, pl.BlockSpec((bm, C), lambda i: (i, 0)))
    return pl.pallas_call(
        functools.partial(_epi_body, relu=relu, emit_norm=emit_norm),
        grid=(N // bm,),
        in_specs=[
            pl.BlockSpec((bm, C), lambda i: (i, 0)),
            pl.BlockSpec((bm, C), lambda i: (i, 0)),
            pl.BlockSpec((bm, 1), lambda i: (i, 0)),
            pl.BlockSpec((1, C), lambda i: (0, 0)),
        ],
        out_specs=out_specs,
        out_shape=out_shape,
    )(a0, xwp, dinv_col, b.reshape(1, C))


_hist_plain = _sc_hist(weighted=False)
_hist_w = _sc_hist(weighted=True)
_msg_plain = _sc_msg(weighted=False)
_msg_w_sc = _sc_msg(weighted=True)


def _jnp_msg(xw, src, dst, w=None):
    m = xw[src]
    if w is not None:
        m = m * w[:, None]
    return jnp.zeros((N, C), jnp.float32).at[dst].add(m)


def _jnp_edge(A, B, src, dst):
    a = A[src]
    b = B[dst]
    sim = jnp.sum(a[:, :C] * b[:, :C], axis=1)
    dots = jnp.sum((a[:, C:2 * C] * b[:, C:]).reshape(E, HEADS, HD), axis=-1)
    w = jnp.mean(jnp.exp(dots * INV_SQRT_HD - a[:, 2 * C:2 * C + HEADS]),
                 axis=-1)
    return sim * w


@jax.jit
def kernel(x, edge_index, W1, b1, W2, b2, W3, b3,
           in_proj_w, in_proj_b, out_proj_w, out_proj_b):
    src = edge_index[0]
    dst = edge_index[1]

    hist1 = jnp.zeros((1, N), jnp.float32).at[0, dst].add(1.0)
    dinv1 = _deg_to_dinv(hist1)
    dinv1_col = dinv1.reshape(N, 1)
    xw1p = _matmul(x, W1, rowscale=dinv1_col)
    agg1 = _jnp_msg(xw1p, src, dst)
    x1, x1n = _epilogue(agg1, xw1p, dinv1_col, b1,
                        relu=True, emit_norm=True)

    qk = _matmul(x1, jnp.transpose(in_proj_w[:2 * C]), b=in_proj_b[:2 * C])
    q3 = jnp.transpose(qk[:, :C].reshape(N, HEADS, HD), (1, 0, 2))
    k3 = jnp.transpose(qk[:, C:].reshape(N, HEADS, HD), (1, 0, 2))
    lse = _flash_lse(q3, k3)
    L = jnp.concatenate(
        [jnp.transpose(lse), jnp.full((N, LANE - HEADS), BIG, jnp.float32),
         jnp.zeros((N, 128 - LANE), jnp.float32)], axis=1)
    A = jnp.concatenate([x1n, qk[:, :C], L], axis=1)
    B = jnp.concatenate([x1n, qk[:, C:]], axis=1)
    ew = _jnp_edge(A, B, src, dst)

    hist2 = jnp.zeros((1, N), jnp.float32).at[0, dst].add(ew)
    dinv2 = _deg_to_dinv(hist2)
    dinv2_col = dinv2.reshape(N, 1)

    xw2p = _matmul(x1, W2, rowscale=dinv2_col)
    agg2 = _jnp_msg(xw2p, src, dst, ew)
    x2 = _epilogue(agg2, xw2p, dinv2_col, b2, relu=True)

    xw3p = _matmul(x2, W3, rowscale=dinv2_col)
    agg3 = _jnp_msg(xw3p, src, dst, ew)
    out = _epilogue(agg3, xw3p, dinv2_col, b3, relu=False)
    return out


# speedup vs baseline: 2.2637x; 1.0002x over previous
import functools
import math

import jax
import jax.numpy as jnp
from jax import lax
from jax.experimental import pallas as pl

N = 4096
E = 131072
C = 256
HEADS = 8
HD = 32
LANE = 16
INV_SQRT_HD = 1.0 / math.sqrt(HD)
BIG = 1e30


def _scatter_messages(xw, src, dst, w=None):
    m = xw[src]
    if w is not None:
        m = m * w[:, None]
    return jnp.zeros((N, C), jnp.float32).at[dst].add(m)


def _edge_weights(A, B, src, dst):
    a = A[src]
    b = B[dst]
    sim = jnp.sum(a[:, :C] * b[:, :C], axis=1)
    dots = jnp.sum((a[:, C:2 * C] * b[:, C:]).reshape(E, HEADS, HD), axis=-1)
    w = jnp.mean(jnp.exp(dots * INV_SQRT_HD - a[:, 2 * C:2 * C + HEADS]),
                 axis=-1)
    return sim * w


def _mm_body(x_ref, w_ref, *rest, bias, rowscale):
    y = jnp.dot(x_ref[...], w_ref[...], preferred_element_type=jnp.float32)
    i = 0
    if bias:
        y = y + rest[i][...]
        i += 1
    if rowscale:
        y = y * rest[i][...]
        i += 1
    rest[i][...] = y


def _matmul(x, w, b=None, rowscale=None, bm=512):
    m, k = x.shape
    n = w.shape[1]
    args = [x, w]
    in_specs = [
        pl.BlockSpec((bm, k), lambda i: (i, 0)),
        pl.BlockSpec((k, n), lambda i: (0, 0)),
    ]
    if b is not None:
        args.append(b.reshape(1, n))
        in_specs.append(pl.BlockSpec((1, n), lambda i: (0, 0)))
    if rowscale is not None:
        args.append(rowscale)
        in_specs.append(pl.BlockSpec((bm, 1), lambda i: (i, 0)))
    return pl.pallas_call(
        functools.partial(_mm_body, bias=b is not None,
                          rowscale=rowscale is not None),
        grid=(m // bm,),
        in_specs=in_specs,
        out_specs=pl.BlockSpec((bm, n), lambda i: (i, 0)),
        out_shape=jax.ShapeDtypeStruct((m, n), jnp.float32),
    )(*args)


def _lse_body(q_ref, k_ref, o_ref, *, bq):
    i = pl.program_id(1)
    q = q_ref[0]
    k = k_ref[0]
    s = lax.dot_general(q, k, (((1,), (1,)), ((), ())),
                        preferred_element_type=jnp.float32) * INV_SQRT_HD
    m = jnp.max(s, axis=1, keepdims=True)
    lse = m[:, 0] + jnp.log(jnp.sum(jnp.exp(s - m), axis=1))
    o_ref[0, 0, pl.ds(i * bq, bq)] = lse


def _flash_lse(q3, k3, bq=256):
    out = pl.pallas_call(
        functools.partial(_lse_body, bq=bq),
        grid=(HEADS, N // bq),
        in_specs=[
            pl.BlockSpec((1, bq, HD), lambda h, i: (h, i, 0)),
            pl.BlockSpec((1, N, HD), lambda h, i: (h, 0, 0)),
        ],
        out_specs=pl.BlockSpec((1, 1, N), lambda h, i: (h, 0, 0)),
        out_shape=jax.ShapeDtypeStruct((HEADS, 1, N), jnp.float32),
    )(q3, k3)
    return out.reshape(HEADS, N)


def _deg_body(h_ref, o_ref):
    deg = 1.0 + jnp.sum(h_ref[...], axis=0, keepdims=True)
    o_ref[...] = jnp.where(deg > 0, lax.rsqrt(jnp.where(deg > 0, deg, 1.0)),
                           0.0)


def _deg_to_dinv(hist):
    return pl.pallas_call(
        _deg_body,
        in_specs=[pl.BlockSpec((1, N), lambda: (0, 0))],
        out_specs=pl.BlockSpec((1, N), lambda: (0, 0)),
        out_shape=jax.ShapeDtypeStruct((1, N), jnp.float32),
    )(hist)


def _epi_body(a0_ref, xwp_ref, dinv_ref, b_ref, o_ref, *norm_out,
              relu, emit_norm):
    y = dinv_ref[...] * (a0_ref[...] + xwp_ref[...]) + b_ref[...]
    if relu:
        y = jnp.maximum(y, 0.0)
    o_ref[...] = y
    if emit_norm:
        n2 = jnp.sum(y * y, axis=1, keepdims=True)
        inv = 1.0 / jnp.maximum(jnp.sqrt(n2), 1e-8)
        norm_out[0][...] = y * inv


def _epilogue(a0, xwp, dinv_col, b, relu, emit_norm=False, bm=512):
    out_shape = jax.ShapeDtypeStruct((N, C), jnp.float32)
    out_specs = pl.BlockSpec((bm, C), lambda i: (i, 0))
    if emit_norm:
        out_shape = (out_shape, jax.ShapeDtypeStruct((N, C), jnp.float32))
        out_specs = (out_specs, pl.BlockSpec((bm, C), lambda i: (i, 0)))
    return pl.pallas_call(
        functools.partial(_epi_body, relu=relu, emit_norm=emit_norm),
        grid=(N // bm,),
        in_specs=[
            pl.BlockSpec((bm, C), lambda i: (i, 0)),
            pl.BlockSpec((bm, C), lambda i: (i, 0)),
            pl.BlockSpec((bm, 1), lambda i: (i, 0)),
            pl.BlockSpec((1, C), lambda i: (0, 0)),
        ],
        out_specs=out_specs,
        out_shape=out_shape,
    )(a0, xwp, dinv_col, b.reshape(1, C))


@jax.jit
def kernel(x, edge_index, W1, b1, W2, b2, W3, b3,
           in_proj_w, in_proj_b, out_proj_w, out_proj_b):
    src = edge_index[0]
    dst = edge_index[1]

    hist1 = jnp.zeros((1, N), jnp.float32).at[0, dst].add(1.0)
    dinv1 = _deg_to_dinv(hist1)
    dinv1_col = dinv1.reshape(N, 1)
    xw1p = _matmul(x, W1, rowscale=dinv1_col)
    agg1 = _scatter_messages(xw1p, src, dst)
    x1, x1n = _epilogue(agg1, xw1p, dinv1_col, b1,
                        relu=True, emit_norm=True)

    qk = _matmul(x1, jnp.transpose(in_proj_w[:2 * C]), b=in_proj_b[:2 * C])
    q3 = jnp.transpose(qk[:, :C].reshape(N, HEADS, HD), (1, 0, 2))
    k3 = jnp.transpose(qk[:, C:].reshape(N, HEADS, HD), (1, 0, 2))
    lse = _flash_lse(q3, k3)
    L = jnp.concatenate(
        [jnp.transpose(lse), jnp.full((N, LANE - HEADS), BIG, jnp.float32),
         jnp.zeros((N, 128 - LANE), jnp.float32)], axis=1)
    A = jnp.concatenate([x1n, qk[:, :C], L], axis=1)
    B = jnp.concatenate([x1n, qk[:, C:]], axis=1)
    ew = _edge_weights(A, B, src, dst)

    hist2 = jnp.zeros((1, N), jnp.float32).at[0, dst].add(ew)
    dinv2 = _deg_to_dinv(hist2)
    dinv2_col = dinv2.reshape(N, 1)

    xw2p = _matmul(x1, W2, rowscale=dinv2_col)
    agg2 = _scatter_messages(xw2p, src, dst, ew)
    x2 = _epilogue(agg2, xw2p, dinv2_col, b2, relu=True)

    xw3p = _matmul(x2, W3, rowscale=dinv2_col)
    agg3 = _scatter_messages(xw3p, src, dst, ew)
    out = _epilogue(agg3, xw3p, dinv2_col, b3, relu=False)
    return out
